# baseline (speedup 1.0000x reference)
import functools

import jax
import jax.numpy as jnp
from jax.experimental import pallas as pl
from jax.experimental.pallas import tpu as pltpu
from jax.experimental.pallas import tpu_sc as plsc


def _nn_body(q_ref, k_ref, dmin_ref, imin_ref, mean_ref,
             qm2_ref, qsq_ref, accd_ref, acci_ref, *, nblocks, bk, nq):
    i = pl.program_id(0)
    ng = bk // 8

    @pl.when(i == 0)
    def _init():
        accd_ref[...] = jnp.full((8, nq), jnp.inf, jnp.float32)
        acci_ref[...] = jnp.zeros((8, nq), jnp.int32)
        qn = q_ref[...] / 224.0
        qm2_ref[...] = qn * -2.0
        qn2 = qn * qn
        ones = jnp.ones((1, qn.shape[1]), jnp.float32)
        qsq_ref[...] = jax.lax.dot_general(
            ones, qn2, (((1,), (1,)), ((), ())),
            precision=jax.lax.Precision.HIGHEST)

    kn = k_ref[...] / 224.0
    ksq = jnp.sum(kn * kn, axis=1, keepdims=True)
    mm2 = jax.lax.dot_general(kn, qm2_ref[...], (((1,), (1,)), ((), ())),
                              preferred_element_type=jnp.float32)
    s = qsq_ref[...] + ksq
    d = s + mm2
    d3 = d.reshape(ng, 8, nq)
    bmin = jnp.min(d3, axis=0)
    bidx = jnp.argmin(d3, axis=0)

    upd = bmin < accd_ref[...]
    acci_ref[...] = jnp.where(upd, bidx + (i * ng), acci_ref[...])
    accd_ref[...] = jnp.where(upd, bmin, accd_ref[...])

    @pl.when(i == nblocks - 1)
    def _fin():
        ad = accd_ref[...]
        fmin = jnp.min(ad, axis=0, keepdims=True)
        rowid = acci_ref[...] * 8 + jax.lax.broadcasted_iota(
            jnp.int32, (8, nq), 0)
        fidx = jnp.min(jnp.where(ad == fmin, rowid, 0x7FFFFFFF),
                       axis=0, keepdims=True)
        dmin_ref[...] = fmin
        imin_ref[...] = fidx
        mean_ref[...] = (jnp.sum(fmin) / nq).reshape(1, 1)


def _pick_block(nk):
    for c in (5000, 4000, 2000, 2048, 1600, 1024, 800, 512, 400, 256, 200,
              128, 80, 64, 40, 16, 8):
        if nk % c == 0:
            return c
    return nk


def _distance_argmin(queries, keys):
    nq, nd = queries.shape
    nk = keys.shape[0]
    bk = _pick_block(nk)
    nblocks = nk // bk
    body = functools.partial(_nn_body, nblocks=nblocks, bk=bk, nq=nq)
    dmin, imin, mean = pl.pallas_call(
        body,
        grid=(nblocks,),
        in_specs=[
            pl.BlockSpec((nq, nd), lambda i: (0, 0)),
            pl.BlockSpec((bk, nd), lambda i: (i, 0)),
        ],
        out_specs=[
            pl.BlockSpec((1, nq), lambda i: (0, 0)),
            pl.BlockSpec((1, nq), lambda i: (0, 0)),
            pl.BlockSpec((1, 1), lambda i: (0, 0)),
        ],
        out_shape=[
            jax.ShapeDtypeStruct((1, nq), jnp.float32),
            jax.ShapeDtypeStruct((1, nq), jnp.int32),
            jax.ShapeDtypeStruct((1, 1), jnp.float32),
        ],
        scratch_shapes=[
            pltpu.VMEM((nq, nd), jnp.float32),
            pltpu.VMEM((1, nq), jnp.float32),
            pltpu.VMEM((8, nq), jnp.float32),
            pltpu.VMEM((8, nq), jnp.int32),
        ],
    )(queries, keys)
    return dmin, imin, mean


def _sc_gather(keys, idx_row):
    nq = idx_row.shape[1]
    nd = keys.shape[1]
    window = 128
    mesh = plsc.VectorSubcoreMesh(core_axis_name="core",
                                  subcore_axis_name="subcore")

    @functools.partial(
        pl.kernel,
        out_type=jax.ShapeDtypeStruct((nq, nd), keys.dtype),
        mesh=mesh,
    )
    def gather_kernel(keys_hbm, idx_hbm, out_hbm):
        def body(i_vmem, o_vmem):
            pltpu.sync_copy(keys_hbm.at[i_vmem.at[0]], o_vmem)

        pltpu.emit_pipeline(
            body,
            grid=(nq // window,),
            in_specs=[pl.BlockSpec((1, window), lambda i: (0, i))],
            out_specs=[pl.BlockSpec((window, nd), lambda i: (i, 0))],
            core_axis_name=("core", "subcore"),
            dimension_semantics=(pltpu.PARALLEL,),
        )(idx_hbm, out_hbm)

    return gather_kernel(keys, idx_row)


def kernel(queries, keys):
    nq = queries.shape[0]
    dmin, imin, mean = _distance_argmin(queries, keys)
    nn_points = _sc_gather(keys, imin)
    dists = dmin.reshape(nq, 1)
    nn_idx = imin.reshape(nq, 1)
    return dists, nn_idx, nn_points[:, None, :], mean.reshape(())


# speedup vs baseline: 1.0170x; 1.0170x over previous
import functools

import jax
import jax.numpy as jnp
from jax.experimental import pallas as pl
from jax.experimental.pallas import tpu as pltpu
from jax.experimental.pallas import tpu_sc as plsc


def _nn_body(q_ref, k_ref, dmin_ref, imin_ref, mean_ref,
             qm2_ref, qsq_ref, accd_ref, acci_ref, *, nblocks, bk, nq):
    i = pl.program_id(0)
    ng = bk // 8

    @pl.when(i == 0)
    def _init():
        accd_ref[...] = jnp.full((8, nq), jnp.inf, jnp.float32)
        acci_ref[...] = jnp.zeros((8, nq), jnp.int32)
        qn = q_ref[...] / 224.0
        qm2_ref[...] = qn * -2.0
        qn2 = qn * qn
        ones = jnp.ones((1, qn.shape[1]), jnp.float32)
        qsq_ref[...] = jax.lax.dot_general(
            ones, qn2, (((1,), (1,)), ((), ())),
            precision=jax.lax.Precision.HIGHEST)

    kn = k_ref[...] / 224.0
    ksq = jnp.sum(kn * kn, axis=1, keepdims=True)
    mm2 = jax.lax.dot_general(kn, qm2_ref[...], (((1,), (1,)), ((), ())),
                              preferred_element_type=jnp.float32)
    s = qsq_ref[...] + ksq
    d = s + mm2
    d3 = d.reshape(ng, 8, nq)
    bmin = jnp.min(d3, axis=0)
    bidx = jnp.argmin(d3, axis=0)

    upd = bmin < accd_ref[...]
    acci_ref[...] = jnp.where(upd, bidx + (i * ng), acci_ref[...])
    accd_ref[...] = jnp.where(upd, bmin, accd_ref[...])

    @pl.when(i == nblocks - 1)
    def _fin():
        ad = accd_ref[...]
        fmin = jnp.min(ad, axis=0, keepdims=True)
        rowid = acci_ref[...] * 8 + jax.lax.broadcasted_iota(
            jnp.int32, (8, nq), 0)
        fidx = jnp.min(jnp.where(ad == fmin, rowid, 0x7FFFFFFF),
                       axis=0, keepdims=True)
        dmin_ref[...] = fmin
        imin_ref[...] = fidx
        mean_ref[...] = (jnp.sum(fmin) / nq).reshape(1, 1)


def _pick_block(nk):
    for c in (10000, 5000, 4000, 2000, 2048, 1600, 1024, 800, 512, 400, 256,
              200, 128, 80, 64, 40, 16, 8):
        if nk % c == 0:
            return c
    return nk


def _distance_argmin(queries, keys):
    nq, nd = queries.shape
    nk = keys.shape[0]
    bk = _pick_block(nk)
    nblocks = nk // bk
    body = functools.partial(_nn_body, nblocks=nblocks, bk=bk, nq=nq)
    dmin, imin, mean = pl.pallas_call(
        body,
        grid=(nblocks,),
        in_specs=[
            pl.BlockSpec((nq, nd), lambda i: (0, 0)),
            pl.BlockSpec((bk, nd), lambda i: (i, 0)),
        ],
        out_specs=[
            pl.BlockSpec((1, nq), lambda i: (0, 0)),
            pl.BlockSpec((1, nq), lambda i: (0, 0)),
            pl.BlockSpec((1, 1), lambda i: (0, 0)),
        ],
        out_shape=[
            jax.ShapeDtypeStruct((1, nq), jnp.float32),
            jax.ShapeDtypeStruct((1, nq), jnp.int32),
            jax.ShapeDtypeStruct((1, 1), jnp.float32),
        ],
        scratch_shapes=[
            pltpu.VMEM((nq, nd), jnp.float32),
            pltpu.VMEM((1, nq), jnp.float32),
            pltpu.VMEM((8, nq), jnp.float32),
            pltpu.VMEM((8, nq), jnp.int32),
        ],
    )(queries, keys)
    return dmin, imin, mean


def _sc_gather(keys, idx_row):
    nq = idx_row.shape[1]
    nd = keys.shape[1]
    window = 128
    mesh = plsc.VectorSubcoreMesh(core_axis_name="core",
                                  subcore_axis_name="subcore")

    @functools.partial(
        pl.kernel,
        out_type=jax.ShapeDtypeStruct((nq, nd), keys.dtype),
        mesh=mesh,
    )
    def gather_kernel(keys_hbm, idx_hbm, out_hbm):
        def body(i_vmem, o_vmem):
            pltpu.sync_copy(keys_hbm.at[i_vmem.at[0]], o_vmem)

        pltpu.emit_pipeline(
            body,
            grid=(nq // window,),
            in_specs=[pl.BlockSpec((1, window), lambda i: (0, i))],
            out_specs=[pl.BlockSpec((window, nd), lambda i: (i, 0))],
            core_axis_name=("core", "subcore"),
            dimension_semantics=(pltpu.PARALLEL,),
        )(idx_hbm, out_hbm)

    return gather_kernel(keys, idx_row)


def kernel(queries, keys):
    nq = queries.shape[0]
    dmin, imin, mean = _distance_argmin(queries, keys)
    nn_points = _sc_gather(keys, imin)
    dists = dmin.reshape(nq, 1)
    nn_idx = imin.reshape(nq, 1)
    return dists, nn_idx, nn_points[:, None, :], mean.reshape(())
